# baseline (speedup 1.0000x reference)
import functools

import jax
import jax.numpy as jnp
from jax import lax
from jax.experimental import pallas as pl
from jax.experimental.pallas import tpu as pltpu
from jax.experimental.pallas import tpu_sc as plsc

N = 10000
G = 256
EG = 320000
SDIM = 256
NA = 16
NB = 5

NBLK = 256
NPAD = 10240
NNB = NPAD // NBLK

NC = 2
NS = 16
NW = NC * NS
EB = 64
EPW = 10240
EGP = NW * EPW
ECH = EPW // EB
NPAIR = ECH // 2
TBW = 512
TBWI = TBW // 2

EBLK = 2048
NEB = EGP // EBLK


def _mean_body(pos_ref, batch_ref, mean_ref, acc_ref):
    i = pl.program_id(0)
    bcol = batch_ref[...]
    gids = lax.broadcasted_iota(jnp.int32, (NBLK, G), 1)
    onehot = (bcol == gids).astype(jnp.float32)
    ext = jnp.concatenate(
        [pos_ref[...], jnp.ones((NBLK, 1), jnp.float32)], axis=1)
    contrib = lax.dot_general(onehot, ext, (((0,), (0,)), ((), ())),
                              preferred_element_type=jnp.float32)

    @pl.when(i == 0)
    def _():
        acc_ref[...] = contrib

    @pl.when(i > 0)
    def _():
        acc_ref[...] = acc_ref[...] + contrib

    @pl.when(i == NNB - 1)
    def _():
        acc = acc_ref[...]
        mean_ref[...] = acc / jnp.maximum(acc[:, 3:4], 1.0)


def _graph_mean(pos_pad, batch_col):
    return pl.pallas_call(
        _mean_body,
        grid=(NNB,),
        in_specs=[
            pl.BlockSpec((NBLK, 3), lambda i: (i, 0)),
            pl.BlockSpec((NBLK, 1), lambda i: (i, 0)),
        ],
        out_specs=pl.BlockSpec((G, 4), lambda i: (0, 0)),
        out_shape=jax.ShapeDtypeStruct((G, 4), jnp.float32),
        scratch_shapes=[pltpu.VMEM((G, 4), jnp.float32)],
    )(pos_pad, batch_col)


def _node_body(pos_ref, batch_ref, x_ref, mean_ref, t_ref,
               wti_ref, bti_ref, wat_ref, bat_ref, wa2_ref, ba2_ref,
               wsh_ref, bsh_ref, wb0_ref, wal_ref, bal_ref,
               posc_ref, tab_ref, at_ref):
    bcol = batch_ref[...]
    gids = lax.broadcasted_iota(jnp.int32, (NBLK, G), 1)
    onehot = (bcol == gids).astype(jnp.float32)

    mean3 = mean_ref[...][:, 0:3]
    posc = pos_ref[...] - jnp.dot(onehot, mean3,
                                  preferred_element_type=jnp.float32)
    posc_ref[...] = posc

    temb = t_ref[...] * wti_ref[...] + bti_ref[...]
    tnode = jnp.dot(onehot, temb, preferred_element_type=jnp.float32)

    s = jnp.dot(x_ref[...], wat_ref[...],
                preferred_element_type=jnp.float32) + bat_ref[...]
    s = jnp.dot(s + tnode, wa2_ref[...],
                preferred_element_type=jnp.float32) + ba2_ref[...]
    sh = jax.nn.silu(jnp.dot(s, wsh_ref[...],
                             preferred_element_type=jnp.float32) + bsh_ref[...])

    tab_ref[:, 0:SDIM] = jnp.dot(
        sh, wb0_ref[...],
        preferred_element_type=jnp.float32).astype(jnp.bfloat16)
    tab_ref[:, SDIM:TBW] = jnp.concatenate(
        [posc, jnp.zeros((NBLK, TBW - SDIM - 3), jnp.float32)],
        axis=1).astype(jnp.bfloat16)
    at_ref[...] = jnp.dot(sh, wal_ref[...],
                          preferred_element_type=jnp.float32) + bal_ref[...]


def _node_dense(pos_pad, batch_col, x_pad, mean4, t,
                W_time, b_time, W_atom, b_atom, W_at, b_at,
                W_sh, b_sh, W_b0c, W_al, b_al):
    full = lambda r, c: pl.BlockSpec((r, c), lambda i: (0, 0))
    return pl.pallas_call(
        _node_body,
        grid=(NNB,),
        in_specs=[
            pl.BlockSpec((NBLK, 3), lambda i: (i, 0)),
            pl.BlockSpec((NBLK, 1), lambda i: (i, 0)),
            pl.BlockSpec((NBLK, NA), lambda i: (i, 0)),
            full(G, 4), full(G, 1),
            full(1, SDIM), full(1, SDIM),
            full(NA, SDIM), full(1, SDIM),
            full(SDIM, SDIM), full(1, SDIM),
            full(SDIM, SDIM), full(1, SDIM),
            full(SDIM, SDIM),
            full(SDIM, 2 * NA), full(1, 2 * NA),
        ],
        out_specs=[
            pl.BlockSpec((NBLK, 3), lambda i: (i, 0)),
            pl.BlockSpec((NBLK, TBW), lambda i: (i, 0)),
            pl.BlockSpec((NBLK, 2 * NA), lambda i: (i, 0)),
        ],
        out_shape=[
            jax.ShapeDtypeStruct((NPAD, 3), jnp.float32),
            jax.ShapeDtypeStruct((NPAD, TBW), jnp.bfloat16),
            jax.ShapeDtypeStruct((NPAD, 2 * NA), jnp.float32),
        ],
    )(pos_pad, batch_col, x_pad, mean4, t,
      W_time.reshape(1, SDIM), b_time.reshape(1, SDIM),
      W_atom, b_atom.reshape(1, SDIM), W_at, b_at.reshape(1, SDIM),
      W_sh, b_sh.reshape(1, SDIM), W_b0c, W_al, b_al.reshape(1, 2 * NA))


GBYTES = 2 * EB * TBWI * 4
WBYTES = EB * TBWI * 4


def _edge_sc_body(cidx_hbm, tab_hbm, out_hbm,
                  idx_v, rows_a, rows_b, semga, semgb, semwa, semwb):
    wid = lax.axis_index("s") * NC + lax.axis_index("c")
    pltpu.sync_copy(cidx_hbm.at[pl.ds(wid * 2 * EPW, 2 * EPW)], idx_v)
    ebase = wid * EPW

    def fire_gather(t, buf, sem):
        pltpu.async_copy(tab_hbm.at[idx_v.at[pl.ds(t * 2 * EB, 2 * EB)]],
                         buf, sem)

    def wait_gather(buf, sem):
        pltpu.make_async_copy(
            tab_hbm.at[idx_v.at[pl.ds(0, 2 * EB)]], buf, sem).wait()

    def wait_write(buf, sem):
        pltpu.make_async_copy(
            buf.at[pl.ds(0, EB)], out_hbm.at[pl.ds(ebase, EB)], sem).wait()

    def compute(buf):
        @plsc.parallel_loop(0, EB, unroll=2)
        def _(e):
            for k in range(SDIM // 32):
                sl = pl.ds(k * 16, 16)
                a = plsc.bitcast(buf[e, sl], jnp.bfloat16)
                b = plsc.bitcast(buf[EB + e, sl], jnp.bfloat16)
                buf[e, sl] = plsc.bitcast(a + b, jnp.int32)
            psl = pl.ds(SDIM // 2, 16)
            a = plsc.bitcast(buf[e, psl], jnp.bfloat16)
            b = plsc.bitcast(buf[EB + e, psl], jnp.bfloat16)
            buf[e, psl] = plsc.bitcast(a - b, jnp.int32)

    def write_out(t, buf, sem):
        pltpu.async_copy(buf.at[pl.ds(0, EB)],
                         out_hbm.at[pl.ds(ebase + t * EB, EB)], sem)

    fire_gather(0, rows_a, semga)

    def pair(tt, carry):
        t0 = 2 * tt
        t1 = t0 + 1

        @pl.when(tt > 0)
        def _():
            wait_write(rows_b, semwb)
        fire_gather(t1, rows_b, semgb)
        wait_gather(rows_a, semga)
        compute(rows_a)
        write_out(t0, rows_a, semwa)
        wait_gather(rows_b, semgb)
        wait_write(rows_a, semwa)

        @pl.when(tt < NPAIR - 1)
        def _():
            fire_gather(t0 + 2, rows_a, semga)
        compute(rows_b)
        write_out(t1, rows_b, semwb)
        return carry

    lax.fori_loop(0, NPAIR, pair, 0)
    wait_write(rows_b, semwb)


def _edge_gather(cidx, tab):
    mesh = plsc.VectorSubcoreMesh(core_axis_name="c", subcore_axis_name="s")
    f = functools.partial(
        pl.kernel, _edge_sc_body, mesh=mesh,
        compiler_params=pltpu.CompilerParams(needs_layout_passes=False),
        out_type=jax.ShapeDtypeStruct((EGP, TBWI), jnp.int32),
        scratch_types=[
            pltpu.VMEM((2 * EPW,), jnp.int32),
            pltpu.VMEM((2 * EB, TBWI), jnp.int32),
            pltpu.VMEM((2 * EB, TBWI), jnp.int32),
            pltpu.SemaphoreType.DMA,
            pltpu.SemaphoreType.DMA,
            pltpu.SemaphoreType.DMA,
            pltpu.SemaphoreType.DMA,
        ],
    )()
    return f(cidx, tab)


def _bond_body(wide_ref, wd_ref, bb0_ref, wb1_ref, bb1_ref, out_ref):
    blk = wide_ref[...].astype(jnp.float32)
    fsum = blk[:, 0:SDIM]
    pd = blk[:, SDIM:SDIM + 16]
    d = jnp.sqrt(jnp.sum(pd * pd, axis=1, keepdims=True))
    h = fsum + d * wd_ref[...] + bb0_ref[...]
    hs = jax.nn.silu(h)
    out_ref[...] = jnp.dot(hs, wb1_ref[...],
                           preferred_element_type=jnp.float32) + bb1_ref[...]


def _bond_tail(wide, wd, bb0, wb1p, bb1p):
    full = lambda r, c: pl.BlockSpec((r, c), lambda i: (0, 0))
    return pl.pallas_call(
        _bond_body,
        grid=(NEB,),
        in_specs=[
            pl.BlockSpec((EBLK, TBW), lambda i: (i, 0)),
            full(1, SDIM), full(1, SDIM),
            full(SDIM, 16), full(1, 16),
        ],
        out_specs=pl.BlockSpec((EBLK, 16), lambda i: (i, 0)),
        out_shape=jax.ShapeDtypeStruct((EGP, 16), jnp.float32),
    )(wide, wd, bb0, wb1p, bb1p)


def kernel(x, t, pos, edge_index_local, edge_index_global, edge_attr_global,
           batch, batch_edge_global, W_time, b_time, W_atom, b_atom, W_at,
           b_at, W_sh, b_sh, W_b0, b_b0, W_b1, b_b1, W_co, W_al, b_al):
    pos_pad = jnp.pad(pos, ((0, NPAD - N), (0, 0)))
    x_pad = jnp.pad(x, ((0, NPAD - N), (0, 0)))
    batch_col = jnp.pad(batch, (0, NPAD - N),
                        constant_values=G).reshape(NPAD, 1)

    mean4 = _graph_mean(pos_pad, batch_col)
    posc_pad, tab, at = _node_dense(
        pos_pad, batch_col, x_pad, mean4, t,
        W_time, b_time, W_atom, b_atom, W_at, b_at,
        W_sh, b_sh, W_b0[:SDIM, :], W_al, b_al)

    src_idx = jnp.pad(edge_index_global[0], (0, EGP - EG))
    dst_idx = jnp.pad(edge_index_global[1], (0, EGP - EG))
    cht = EGP // EB
    cidx = jnp.stack([src_idx.reshape(cht, EB),
                      dst_idx.reshape(cht, EB)], axis=1).reshape(2 * EGP)

    tab_i32 = jax.lax.bitcast_convert_type(
        tab.reshape(NPAD, TBWI, 2), jnp.int32)
    wide_i32 = _edge_gather(cidx, tab_i32)
    wide = jax.lax.bitcast_convert_type(
        wide_i32, jnp.bfloat16).reshape(EGP, TBW)

    bonds = _bond_tail(wide,
                       W_b0[SDIM:SDIM + 1, :], b_b0.reshape(1, SDIM),
                       jnp.pad(W_b1, ((0, 0), (0, 16 - 2 * NB))),
                       jnp.pad(b_b1, (0, 16 - 2 * NB)).reshape(1, 16))

    pos_c = posc_pad[:N]
    coords_pred = pos_c
    coords_eps = jnp.zeros((N, 3), jnp.float32)
    atoms_eps = at[:N, :NA]
    atoms_pred = at[:N, NA:]
    bonds_pred = bonds[:EG, :NB]
    bonds_eps = bonds[:EG, NB:2 * NB]
    return (coords_pred, coords_eps, atoms_pred, atoms_eps,
            bonds_pred, bonds_eps, pos_c, x, edge_attr_global)


# speedup vs baseline: 2.5151x; 2.5151x over previous
import functools

import jax
import jax.numpy as jnp
from jax import lax
from jax.experimental import pallas as pl
from jax.experimental.pallas import tpu as pltpu
from jax.experimental.pallas import tpu_sc as plsc

N = 10000
G = 256
EG = 320000
SDIM = 256
NA = 16
NB = 5

NBLK = 256
NPAD = 10240
NNB = NPAD // NBLK

NC = 2
NS = 16
NW = NC * NS
EB = 64
EPW = 10240
EGP = NW * EPW
ECH = EPW // EB
NPAIR = ECH // 2
TBWI = 128

EBLK = 2048
NEB = EGP // EBLK


def _mean_body(pos_ref, batch_ref, mean_ref, acc_ref):
    i = pl.program_id(0)
    bcol = batch_ref[...]
    gids = lax.broadcasted_iota(jnp.int32, (NBLK, G), 1)
    onehot = (bcol == gids).astype(jnp.float32)
    ext = jnp.concatenate(
        [pos_ref[...], jnp.ones((NBLK, 1), jnp.float32)], axis=1)
    contrib = lax.dot_general(onehot, ext, (((0,), (0,)), ((), ())),
                              preferred_element_type=jnp.float32)

    @pl.when(i == 0)
    def _():
        acc_ref[...] = contrib

    @pl.when(i > 0)
    def _():
        acc_ref[...] = acc_ref[...] + contrib

    @pl.when(i == NNB - 1)
    def _():
        acc = acc_ref[...]
        mean_ref[...] = acc / jnp.maximum(acc[:, 3:4], 1.0)


def _graph_mean(pos_pad, batch_col):
    return pl.pallas_call(
        _mean_body,
        grid=(NNB,),
        in_specs=[
            pl.BlockSpec((NBLK, 3), lambda i: (i, 0)),
            pl.BlockSpec((NBLK, 1), lambda i: (i, 0)),
        ],
        out_specs=pl.BlockSpec((G, 4), lambda i: (0, 0)),
        out_shape=jax.ShapeDtypeStruct((G, 4), jnp.float32),
        scratch_shapes=[pltpu.VMEM((G, 4), jnp.float32)],
    )(pos_pad, batch_col)


def _pack_bf16_pair(a, b):
    ba = jax.lax.bitcast_convert_type(a, jnp.int32)
    bb = jax.lax.bitcast_convert_type(b, jnp.int32)
    ra = ba + 0x7FFF + ((ba >> 16) & 1)
    rb = bb + 0x7FFF + ((bb >> 16) & 1)
    return (ra & jnp.int32(-65536)) | ((rb >> 16) & jnp.int32(0xFFFF))


def _node_body(pos_ref, batch_ref, x_ref, mean_ref, t_ref,
               wti_ref, bti_ref, wat_ref, bat_ref, wa2_ref, ba2_ref,
               wsh_ref, bsh_ref, wb0_ref, wal_ref, bal_ref,
               posc_ref, tab_ref, at_ref):
    bcol = batch_ref[...]
    gids = lax.broadcasted_iota(jnp.int32, (NBLK, G), 1)
    onehot = (bcol == gids).astype(jnp.float32)

    mean3 = mean_ref[...][:, 0:3]
    posc = pos_ref[...] - jnp.dot(onehot, mean3,
                                  preferred_element_type=jnp.float32)
    posc_ref[...] = posc

    temb = t_ref[...] * wti_ref[...] + bti_ref[...]
    tnode = jnp.dot(onehot, temb, preferred_element_type=jnp.float32)

    s = jnp.dot(x_ref[...], wat_ref[...],
                preferred_element_type=jnp.float32) + bat_ref[...]
    s = jnp.dot(s + tnode, wa2_ref[...],
                preferred_element_type=jnp.float32) + ba2_ref[...]
    sh = jax.nn.silu(jnp.dot(s, wsh_ref[...],
                             preferred_element_type=jnp.float32) + bsh_ref[...])

    u = jnp.dot(sh, wb0_ref[...], preferred_element_type=jnp.float32)
    tab_ref[...] = _pack_bf16_pair(u[:, 0:TBWI], u[:, TBWI:SDIM])
    at_ref[...] = jnp.dot(sh, wal_ref[...],
                          preferred_element_type=jnp.float32) + bal_ref[...]


def _node_dense(pos_pad, batch_col, x_pad, mean4, t,
                W_time, b_time, W_atom, b_atom, W_at, b_at,
                W_sh, b_sh, W_b0c, W_al, b_al):
    full = lambda r, c: pl.BlockSpec((r, c), lambda i: (0, 0))
    return pl.pallas_call(
        _node_body,
        grid=(NNB,),
        in_specs=[
            pl.BlockSpec((NBLK, 3), lambda i: (i, 0)),
            pl.BlockSpec((NBLK, 1), lambda i: (i, 0)),
            pl.BlockSpec((NBLK, NA), lambda i: (i, 0)),
            full(G, 4), full(G, 1),
            full(1, SDIM), full(1, SDIM),
            full(NA, SDIM), full(1, SDIM),
            full(SDIM, SDIM), full(1, SDIM),
            full(SDIM, SDIM), full(1, SDIM),
            full(SDIM, SDIM),
            full(SDIM, 2 * NA), full(1, 2 * NA),
        ],
        out_specs=[
            pl.BlockSpec((NBLK, 3), lambda i: (i, 0)),
            pl.BlockSpec((NBLK, TBWI), lambda i: (i, 0)),
            pl.BlockSpec((NBLK, 2 * NA), lambda i: (i, 0)),
        ],
        out_shape=[
            jax.ShapeDtypeStruct((NPAD, 3), jnp.float32),
            jax.ShapeDtypeStruct((NPAD, TBWI), jnp.int32),
            jax.ShapeDtypeStruct((NPAD, 2 * NA), jnp.float32),
        ],
    )(pos_pad, batch_col, x_pad, mean4, t,
      W_time.reshape(1, SDIM), b_time.reshape(1, SDIM),
      W_atom, b_atom.reshape(1, SDIM), W_at, b_at.reshape(1, SDIM),
      W_sh, b_sh.reshape(1, SDIM), W_b0c, W_al, b_al.reshape(1, 2 * NA))


def _edge_sc_body(cidx_hbm, tab_hbm, px_hbm, py_hbm, pz_hbm,
                  usum_hbm, ssq_hbm,
                  idx_v, rows_a, rows_b, ob_a, ob_b, sq_a, sq_b,
                  px_v, py_v, pz_v, semga, semgb, semwa, semwb):
    wid = lax.axis_index("s") * NC + lax.axis_index("c")
    pltpu.sync_copy(cidx_hbm.at[pl.ds(wid * 2 * EPW, 2 * EPW)], idx_v)
    pltpu.sync_copy(px_hbm, px_v)
    pltpu.sync_copy(py_hbm, py_v)
    pltpu.sync_copy(pz_hbm, pz_v)
    ebase = wid * EPW

    def fire_gather(t, buf, sem):
        pltpu.async_copy(tab_hbm.at[idx_v.at[pl.ds(t * 2 * EB, 2 * EB)]],
                         buf, sem)

    def wait_gather(buf, sem):
        pltpu.make_async_copy(
            tab_hbm.at[idx_v.at[pl.ds(0, 2 * EB)]], buf, sem).wait()

    def wait_write(ob, sq, sem):
        pltpu.make_async_copy(ob, usum_hbm.at[pl.ds(ebase, EB)], sem).wait()
        pltpu.make_async_copy(sq, ssq_hbm.at[pl.ds(ebase, EB)], sem).wait()

    def compute(t, buf, ob, sq):
        @plsc.parallel_loop(0, EB, unroll=2)
        def _(e):
            for k in range(TBWI // 16):
                sl = pl.ds(k * 16, 16)
                a = plsc.bitcast(buf[e, sl], jnp.bfloat16)
                b = plsc.bitcast(buf[EB + e, sl], jnp.bfloat16)
                ob[e, sl] = plsc.bitcast(a + b, jnp.int32)

        for g in range(EB // 16):
            si = idx_v[pl.ds(t * 2 * EB + g * 16, 16)]
            di = idx_v[pl.ds(t * 2 * EB + EB + g * 16, 16)]
            dx = plsc.load_gather(px_v, [si]) - plsc.load_gather(px_v, [di])
            dy = plsc.load_gather(py_v, [si]) - plsc.load_gather(py_v, [di])
            dz = plsc.load_gather(pz_v, [si]) - plsc.load_gather(pz_v, [di])
            sq[pl.ds(g * 16, 16)] = dx * dx + dy * dy + dz * dz

    def write_out(t, ob, sq, sem):
        pltpu.async_copy(ob, usum_hbm.at[pl.ds(ebase + t * EB, EB)], sem)
        pltpu.async_copy(sq, ssq_hbm.at[pl.ds(ebase + t * EB, EB)], sem)

    fire_gather(0, rows_a, semga)

    def pair(tt, carry):
        t0 = 2 * tt
        t1 = t0 + 1

        @pl.when(tt > 0)
        def _():
            wait_write(ob_b, sq_b, semwb)
        fire_gather(t1, rows_b, semgb)
        wait_gather(rows_a, semga)
        compute(t0, rows_a, ob_a, sq_a)
        write_out(t0, ob_a, sq_a, semwa)
        wait_gather(rows_b, semgb)
        wait_write(ob_a, sq_a, semwa)

        @pl.when(tt < NPAIR - 1)
        def _():
            fire_gather(t0 + 2, rows_a, semga)
        compute(t1, rows_b, ob_b, sq_b)
        write_out(t1, ob_b, sq_b, semwb)
        return carry

    lax.fori_loop(0, NPAIR, pair, 0)
    wait_write(ob_b, sq_b, semwb)


def _edge_gather(cidx, tab, px, py, pz):
    mesh = plsc.VectorSubcoreMesh(core_axis_name="c", subcore_axis_name="s")
    f = functools.partial(
        pl.kernel, _edge_sc_body, mesh=mesh,
        compiler_params=pltpu.CompilerParams(needs_layout_passes=False),
        out_type=[
            jax.ShapeDtypeStruct((EGP, TBWI), jnp.int32),
            jax.ShapeDtypeStruct((EGP,), jnp.float32),
        ],
        scratch_types=[
            pltpu.VMEM((2 * EPW,), jnp.int32),
            pltpu.VMEM((2 * EB, TBWI), jnp.int32),
            pltpu.VMEM((2 * EB, TBWI), jnp.int32),
            pltpu.VMEM((EB, TBWI), jnp.int32),
            pltpu.VMEM((EB, TBWI), jnp.int32),
            pltpu.VMEM((EB,), jnp.float32),
            pltpu.VMEM((EB,), jnp.float32),
            pltpu.VMEM((NPAD,), jnp.float32),
            pltpu.VMEM((NPAD,), jnp.float32),
            pltpu.VMEM((NPAD,), jnp.float32),
            pltpu.SemaphoreType.DMA,
            pltpu.SemaphoreType.DMA,
            pltpu.SemaphoreType.DMA,
            pltpu.SemaphoreType.DMA,
        ],
    )()
    return f(cidx, tab, px, py, pz)


def _bond_body(usum_ref, ssq_ref, wdh_ref, wdl_ref, bbh_ref, bbl_ref,
               wb1h_ref, wb1l_ref, bb1_ref, out_ref):
    w = usum_ref[...]
    hi = jax.lax.bitcast_convert_type(w & jnp.int32(-65536), jnp.float32)
    lo = jax.lax.bitcast_convert_type(w << 16, jnp.float32)
    d = jnp.sqrt(ssq_ref[...])
    hh = jax.nn.silu(hi + d * wdh_ref[...] + bbh_ref[...])
    hl = jax.nn.silu(lo + d * wdl_ref[...] + bbl_ref[...])
    out_ref[...] = (
        jnp.dot(hh, wb1h_ref[...], preferred_element_type=jnp.float32)
        + jnp.dot(hl, wb1l_ref[...], preferred_element_type=jnp.float32)
        + bb1_ref[...])


def _bond_tail(usum, ssq2, wd, bb0, wb1p, bb1p):
    full = lambda r, c: pl.BlockSpec((r, c), lambda i: (0, 0))
    return pl.pallas_call(
        _bond_body,
        grid=(NEB,),
        in_specs=[
            pl.BlockSpec((EBLK, TBWI), lambda i: (i, 0)),
            pl.BlockSpec((EBLK, 1), lambda i: (i, 0)),
            full(1, TBWI), full(1, TBWI),
            full(1, TBWI), full(1, TBWI),
            full(TBWI, 16), full(TBWI, 16), full(1, 16),
        ],
        out_specs=pl.BlockSpec((EBLK, 16), lambda i: (i, 0)),
        out_shape=jax.ShapeDtypeStruct((EGP, 16), jnp.float32),
    )(usum, ssq2, wd[:, 0:TBWI], wd[:, TBWI:SDIM],
      bb0[:, 0:TBWI], bb0[:, TBWI:SDIM],
      wb1p[0:TBWI, :], wb1p[TBWI:SDIM, :], bb1p)


def kernel(x, t, pos, edge_index_local, edge_index_global, edge_attr_global,
           batch, batch_edge_global, W_time, b_time, W_atom, b_atom, W_at,
           b_at, W_sh, b_sh, W_b0, b_b0, W_b1, b_b1, W_co, W_al, b_al):
    pos_pad = jnp.pad(pos, ((0, NPAD - N), (0, 0)))
    x_pad = jnp.pad(x, ((0, NPAD - N), (0, 0)))
    batch_col = jnp.pad(batch, (0, NPAD - N),
                        constant_values=G).reshape(NPAD, 1)

    mean4 = _graph_mean(pos_pad, batch_col)
    posc_pad, tab, at = _node_dense(
        pos_pad, batch_col, x_pad, mean4, t,
        W_time, b_time, W_atom, b_atom, W_at, b_at,
        W_sh, b_sh, W_b0[:SDIM, :], W_al, b_al)

    src_idx = jnp.pad(edge_index_global[0], (0, EGP - EG))
    dst_idx = jnp.pad(edge_index_global[1], (0, EGP - EG))
    cht = EGP // EB
    cidx = jnp.stack([src_idx.reshape(cht, EB),
                      dst_idx.reshape(cht, EB)], axis=1).reshape(2 * EGP)

    usum, ssq = _edge_gather(cidx, tab, posc_pad[:, 0],
                             posc_pad[:, 1], posc_pad[:, 2])

    bonds = _bond_tail(usum, ssq.reshape(EGP, 1),
                       W_b0[SDIM:SDIM + 1, :], b_b0.reshape(1, SDIM),
                       jnp.pad(W_b1, ((0, 0), (0, 16 - 2 * NB))),
                       jnp.pad(b_b1, (0, 16 - 2 * NB)).reshape(1, 16))

    pos_c = posc_pad[:N]
    coords_pred = pos_c
    coords_eps = jnp.zeros((N, 3), jnp.float32)
    atoms_eps = at[:N, :NA]
    atoms_pred = at[:N, NA:]
    bonds_pred = bonds[:EG, :NB]
    bonds_eps = bonds[:EG, NB:2 * NB]
    return (coords_pred, coords_eps, atoms_pred, atoms_eps,
            bonds_pred, bonds_eps, pos_c, x, edge_attr_global)


# speedup vs baseline: 2.5195x; 1.0017x over previous
import functools

import jax
import jax.numpy as jnp
from jax import lax
from jax.experimental import pallas as pl
from jax.experimental.pallas import tpu as pltpu
from jax.experimental.pallas import tpu_sc as plsc

N = 10000
G = 256
EG = 320000
SDIM = 256
NA = 16
NB = 5

NBLK = 256
NPAD = 10240
NNB = NPAD // NBLK

NC = 2
NS = 16
NW = NC * NS
EB = 64
EPW = 10240
EGP = NW * EPW
ECH = EPW // EB
TBWI = 128
CHT = EGP // EB
CH0 = 240
CH1 = CHT // NS - CH0
CHM = max(CH0, CH1)

EBLK = 2048
NEB = EGP // EBLK


def _mean_body(pos_ref, batch_ref, mean_ref, acc_ref):
    i = pl.program_id(0)
    bcol = batch_ref[...]
    gids = lax.broadcasted_iota(jnp.int32, (NBLK, G), 1)
    onehot = (bcol == gids).astype(jnp.float32)
    ext = jnp.concatenate(
        [pos_ref[...], jnp.ones((NBLK, 1), jnp.float32)], axis=1)
    contrib = lax.dot_general(onehot, ext, (((0,), (0,)), ((), ())),
                              preferred_element_type=jnp.float32)

    @pl.when(i == 0)
    def _():
        acc_ref[...] = contrib

    @pl.when(i > 0)
    def _():
        acc_ref[...] = acc_ref[...] + contrib

    @pl.when(i == NNB - 1)
    def _():
        acc = acc_ref[...]
        mean_ref[...] = acc / jnp.maximum(acc[:, 3:4], 1.0)


def _graph_mean(pos_pad, batch_col):
    return pl.pallas_call(
        _mean_body,
        grid=(NNB,),
        in_specs=[
            pl.BlockSpec((NBLK, 3), lambda i: (i, 0)),
            pl.BlockSpec((NBLK, 1), lambda i: (i, 0)),
        ],
        out_specs=pl.BlockSpec((G, 4), lambda i: (0, 0)),
        out_shape=jax.ShapeDtypeStruct((G, 4), jnp.float32),
        scratch_shapes=[pltpu.VMEM((G, 4), jnp.float32)],
    )(pos_pad, batch_col)


def _pack_bf16_pair(a, b):
    ba = jax.lax.bitcast_convert_type(a, jnp.int32)
    bb = jax.lax.bitcast_convert_type(b, jnp.int32)
    ra = ba + 0x7FFF + ((ba >> 16) & 1)
    rb = bb + 0x7FFF + ((bb >> 16) & 1)
    return (ra & jnp.int32(-65536)) | ((rb >> 16) & jnp.int32(0xFFFF))


def _node_body(pos_ref, batch_ref, x_ref, mean_ref, t_ref,
               wti_ref, bti_ref, wat_ref, bat_ref, wa2_ref, ba2_ref,
               wsh_ref, bsh_ref, wb0_ref, wal_ref, bal_ref,
               posc_ref, tab_ref, at_ref):
    bcol = batch_ref[...]
    gids = lax.broadcasted_iota(jnp.int32, (NBLK, G), 1)
    onehot = (bcol == gids).astype(jnp.float32)

    mean3 = mean_ref[...][:, 0:3]
    posc = pos_ref[...] - jnp.dot(onehot, mean3,
                                  preferred_element_type=jnp.float32)
    posc_ref[...] = posc

    temb = t_ref[...] * wti_ref[...] + bti_ref[...]
    tnode = jnp.dot(onehot, temb, preferred_element_type=jnp.float32)

    s = jnp.dot(x_ref[...], wat_ref[...],
                preferred_element_type=jnp.float32) + bat_ref[...]
    s = jnp.dot(s + tnode, wa2_ref[...],
                preferred_element_type=jnp.float32) + ba2_ref[...]
    sh = jax.nn.silu(jnp.dot(s, wsh_ref[...],
                             preferred_element_type=jnp.float32) + bsh_ref[...])

    u = jnp.dot(sh, wb0_ref[...], preferred_element_type=jnp.float32)
    tab_ref[...] = _pack_bf16_pair(u[:, 0:TBWI], u[:, TBWI:SDIM])
    at_ref[...] = jnp.dot(sh, wal_ref[...],
                          preferred_element_type=jnp.float32) + bal_ref[...]


def _node_dense(pos_pad, batch_col, x_pad, mean4, t,
                W_time, b_time, W_atom, b_atom, W_at, b_at,
                W_sh, b_sh, W_b0c, W_al, b_al):
    full = lambda r, c: pl.BlockSpec((r, c), lambda i: (0, 0))
    return pl.pallas_call(
        _node_body,
        grid=(NNB,),
        in_specs=[
            pl.BlockSpec((NBLK, 3), lambda i: (i, 0)),
            pl.BlockSpec((NBLK, 1), lambda i: (i, 0)),
            pl.BlockSpec((NBLK, NA), lambda i: (i, 0)),
            full(G, 4), full(G, 1),
            full(1, SDIM), full(1, SDIM),
            full(NA, SDIM), full(1, SDIM),
            full(SDIM, SDIM), full(1, SDIM),
            full(SDIM, SDIM), full(1, SDIM),
            full(SDIM, SDIM),
            full(SDIM, 2 * NA), full(1, 2 * NA),
        ],
        out_specs=[
            pl.BlockSpec((NBLK, 3), lambda i: (i, 0)),
            pl.BlockSpec((NBLK, TBWI), lambda i: (i, 0)),
            pl.BlockSpec((NBLK, 2 * NA), lambda i: (i, 0)),
        ],
        out_shape=[
            jax.ShapeDtypeStruct((NPAD, 3), jnp.float32),
            jax.ShapeDtypeStruct((NPAD, TBWI), jnp.int32),
            jax.ShapeDtypeStruct((NPAD, 2 * NA), jnp.float32),
        ],
    )(pos_pad, batch_col, x_pad, mean4, t,
      W_time.reshape(1, SDIM), b_time.reshape(1, SDIM),
      W_atom, b_atom.reshape(1, SDIM), W_at, b_at.reshape(1, SDIM),
      W_sh, b_sh.reshape(1, SDIM), W_b0c, W_al, b_al.reshape(1, 2 * NA))


def _edge_sc_body(cidx_hbm, tab_hbm, px_hbm, py_hbm, pz_hbm,
                  usum_hbm, ssq_hbm,
                  idx_v, rows_a, rows_b, ob_a, ob_b, sq_a, sq_b,
                  px_v, py_v, pz_v, semga, semgb, semwa, semwb):
    c = lax.axis_index("c")
    s = lax.axis_index("s")
    mych = lax.select(c == 0, jnp.int32(CH0), jnp.int32(CH1))
    bchunk = lax.select(c == 0, s * CH0, NS * CH0 + s * CH1)
    pltpu.sync_copy(cidx_hbm.at[pl.ds(bchunk * 2 * EB, CHM * 2 * EB)], idx_v)
    pltpu.sync_copy(px_hbm, px_v)
    pltpu.sync_copy(py_hbm, py_v)
    pltpu.sync_copy(pz_hbm, pz_v)
    ebase = bchunk * EB

    def fire_gather(t, buf, sem):
        pltpu.async_copy(tab_hbm.at[idx_v.at[pl.ds(t * 2 * EB, 2 * EB)]],
                         buf, sem)

    def wait_gather(buf, sem):
        pltpu.make_async_copy(
            tab_hbm.at[idx_v.at[pl.ds(0, 2 * EB)]], buf, sem).wait()

    def wait_write(ob, sq, sem):
        pltpu.make_async_copy(ob, usum_hbm.at[pl.ds(ebase, EB)], sem).wait()
        pltpu.make_async_copy(sq, ssq_hbm.at[pl.ds(ebase, EB)], sem).wait()

    def compute(t, buf, ob, sq):
        @plsc.parallel_loop(0, EB, unroll=2)
        def _(e):
            for k in range(TBWI // 16):
                sl = pl.ds(k * 16, 16)
                a = plsc.bitcast(buf[e, sl], jnp.bfloat16)
                b = plsc.bitcast(buf[EB + e, sl], jnp.bfloat16)
                ob[e, sl] = plsc.bitcast(a + b, jnp.int32)

        for g in range(EB // 16):
            si = idx_v[pl.ds(t * 2 * EB + g * 16, 16)]
            di = idx_v[pl.ds(t * 2 * EB + EB + g * 16, 16)]
            dx = plsc.load_gather(px_v, [si]) - plsc.load_gather(px_v, [di])
            dy = plsc.load_gather(py_v, [si]) - plsc.load_gather(py_v, [di])
            dz = plsc.load_gather(pz_v, [si]) - plsc.load_gather(pz_v, [di])
            sq[pl.ds(g * 16, 16)] = dx * dx + dy * dy + dz * dz

    def write_out(t, ob, sq, sem):
        pltpu.async_copy(ob, usum_hbm.at[pl.ds(ebase + t * EB, EB)], sem)
        pltpu.async_copy(sq, ssq_hbm.at[pl.ds(ebase + t * EB, EB)], sem)

    fire_gather(0, rows_a, semga)
    npair = mych // 2

    def pair(tt, carry):
        t0 = 2 * tt
        t1 = t0 + 1

        @pl.when(tt > 0)
        def _():
            wait_write(ob_b, sq_b, semwb)
        fire_gather(t1, rows_b, semgb)
        wait_gather(rows_a, semga)
        compute(t0, rows_a, ob_a, sq_a)
        write_out(t0, ob_a, sq_a, semwa)
        wait_gather(rows_b, semgb)
        wait_write(ob_a, sq_a, semwa)

        @pl.when(tt < npair - 1)
        def _():
            fire_gather(t0 + 2, rows_a, semga)
        compute(t1, rows_b, ob_b, sq_b)
        write_out(t1, ob_b, sq_b, semwb)
        return carry

    lax.fori_loop(0, npair, pair, 0)
    wait_write(ob_b, sq_b, semwb)


def _edge_gather(cidx, tab, px, py, pz):
    mesh = plsc.VectorSubcoreMesh(core_axis_name="c", subcore_axis_name="s")
    f = functools.partial(
        pl.kernel, _edge_sc_body, mesh=mesh,
        compiler_params=pltpu.CompilerParams(needs_layout_passes=False),
        out_type=[
            jax.ShapeDtypeStruct((EGP, TBWI), jnp.int32),
            jax.ShapeDtypeStruct((EGP,), jnp.float32),
        ],
        scratch_types=[
            pltpu.VMEM((CHM * 2 * EB,), jnp.int32),
            pltpu.VMEM((2 * EB, TBWI), jnp.int32),
            pltpu.VMEM((2 * EB, TBWI), jnp.int32),
            pltpu.VMEM((EB, TBWI), jnp.int32),
            pltpu.VMEM((EB, TBWI), jnp.int32),
            pltpu.VMEM((EB,), jnp.float32),
            pltpu.VMEM((EB,), jnp.float32),
            pltpu.VMEM((NPAD,), jnp.float32),
            pltpu.VMEM((NPAD,), jnp.float32),
            pltpu.VMEM((NPAD,), jnp.float32),
            pltpu.SemaphoreType.DMA,
            pltpu.SemaphoreType.DMA,
            pltpu.SemaphoreType.DMA,
            pltpu.SemaphoreType.DMA,
        ],
    )()
    return f(cidx, tab, px, py, pz)


def _bond_body(usum_ref, ssq_ref, wdh_ref, wdl_ref, bbh_ref, bbl_ref,
               wb1h_ref, wb1l_ref, bb1_ref, out_ref):
    w = usum_ref[...]
    hi = jax.lax.bitcast_convert_type(w & jnp.int32(-65536), jnp.float32)
    lo = jax.lax.bitcast_convert_type(w << 16, jnp.float32)
    d = jnp.sqrt(ssq_ref[...])
    hh = jax.nn.silu(hi + d * wdh_ref[...] + bbh_ref[...])
    hl = jax.nn.silu(lo + d * wdl_ref[...] + bbl_ref[...])
    out_ref[...] = (
        jnp.dot(hh, wb1h_ref[...], preferred_element_type=jnp.float32)
        + jnp.dot(hl, wb1l_ref[...], preferred_element_type=jnp.float32)
        + bb1_ref[...])


def _bond_tail(usum, ssq2, wd, bb0, wb1p, bb1p):
    full = lambda r, c: pl.BlockSpec((r, c), lambda i: (0, 0))
    return pl.pallas_call(
        _bond_body,
        grid=(NEB,),
        in_specs=[
            pl.BlockSpec((EBLK, TBWI), lambda i: (i, 0)),
            pl.BlockSpec((EBLK, 1), lambda i: (i, 0)),
            full(1, TBWI), full(1, TBWI),
            full(1, TBWI), full(1, TBWI),
            full(TBWI, 16), full(TBWI, 16), full(1, 16),
        ],
        out_specs=pl.BlockSpec((EBLK, 16), lambda i: (i, 0)),
        out_shape=jax.ShapeDtypeStruct((EGP, 16), jnp.float32),
    )(usum, ssq2, wd[:, 0:TBWI], wd[:, TBWI:SDIM],
      bb0[:, 0:TBWI], bb0[:, TBWI:SDIM],
      wb1p[0:TBWI, :], wb1p[TBWI:SDIM, :], bb1p)


def kernel(x, t, pos, edge_index_local, edge_index_global, edge_attr_global,
           batch, batch_edge_global, W_time, b_time, W_atom, b_atom, W_at,
           b_at, W_sh, b_sh, W_b0, b_b0, W_b1, b_b1, W_co, W_al, b_al):
    pos_pad = jnp.pad(pos, ((0, NPAD - N), (0, 0)))
    x_pad = jnp.pad(x, ((0, NPAD - N), (0, 0)))
    batch_col = jnp.pad(batch, (0, NPAD - N),
                        constant_values=G).reshape(NPAD, 1)

    mean4 = _graph_mean(pos_pad, batch_col)
    posc_pad, tab, at = _node_dense(
        pos_pad, batch_col, x_pad, mean4, t,
        W_time, b_time, W_atom, b_atom, W_at, b_at,
        W_sh, b_sh, W_b0[:SDIM, :], W_al, b_al)

    src_idx = jnp.pad(edge_index_global[0], (0, EGP - EG))
    dst_idx = jnp.pad(edge_index_global[1], (0, EGP - EG))
    cht = EGP // EB
    cidx = jnp.stack([src_idx.reshape(cht, EB),
                      dst_idx.reshape(cht, EB)], axis=1).reshape(2 * EGP)
    cidx = jnp.pad(cidx, (0, CHM * 2 * EB))

    usum, ssq = _edge_gather(cidx, tab, posc_pad[:, 0],
                             posc_pad[:, 1], posc_pad[:, 2])

    bonds = _bond_tail(usum, ssq.reshape(EGP, 1),
                       W_b0[SDIM:SDIM + 1, :], b_b0.reshape(1, SDIM),
                       jnp.pad(W_b1, ((0, 0), (0, 16 - 2 * NB))),
                       jnp.pad(b_b1, (0, 16 - 2 * NB)).reshape(1, 16))

    pos_c = posc_pad[:N]
    coords_pred = pos_c
    coords_eps = jnp.zeros((N, 3), jnp.float32)
    atoms_eps = at[:N, :NA]
    atoms_pred = at[:N, NA:]
    bonds_pred = bonds[:EG, :NB]
    bonds_eps = bonds[:EG, NB:2 * NB]
    return (coords_pred, coords_eps, atoms_pred, atoms_eps,
            bonds_pred, bonds_eps, pos_c, x, edge_attr_global)
